# baseline (speedup 1.0000x reference)
import functools

import jax
import jax.numpy as jnp
from jax import lax
from jax.experimental import pallas as pl
from jax.experimental.pallas import tpu as pltpu
from jax.experimental.pallas import tpu_sc as plsc

_LANES = 16
_BI = 128
_CI = 8
_NBUF = 2
_NW = 32


def _detile_kernel(d, vocab):
    mesh = plsc.VectorSubcoreMesh(core_axis_name="c", subcore_axis_name="s")
    slb = 2
    n_full = vocab // _BI
    n_slab = n_full // slb
    tail = vocab - n_slab * slb * _BI
    tb_w = slb * _BI

    @functools.partial(
        pl.kernel,
        mesh=mesh,
        out_type=jax.ShapeDtypeStruct((vocab // 2, 2 * d), jnp.float32),
        scratch_types=[
            [pltpu.VMEM((d, tb_w), jnp.float32)] * _NBUF,
            [pltpu.VMEM((4, 2, _CI, 2 * d + 5), jnp.float32)] * slb,
            [pltpu.SemaphoreType.DMA] * _NBUF,
            [pltpu.SemaphoreType.DMA] * slb,
        ],
        compiler_params=pltpu.CompilerParams(
            use_tc_tiling_on_sc=True, needs_layout_passes=False
        ),
    )
    def run(tab_hbm, tail_hbm, out_hbm, tb, tt, sg, so):
        cid = lax.axis_index("c")
        sid = lax.axis_index("s")
        w = sid * 2 + cid
        iota = jnp.arange(_LANES, dtype=jnp.int32)

        n_mine = (n_slab - w + _NW - 1) // _NW

        def col0_of(i):
            return pl.multiple_of((i * _NW + w) * slb * _BI, slb * _BI)

        for b in range(_NBUF):
            @pl.when(b < n_mine)
            def _():
                pltpu.async_copy(
                    tab_hbm.at[:, pl.ds(col0_of(b), slb * _BI)],
                    tb[b].at[:, pl.ds(0, slb * _BI)],
                    sg[b],
                )

        def col_body(p, carry):
            for b in range(_NBUF):
                i = p * _NBUF + b

                @pl.when(i < n_mine)
                def _():
                    col0 = col0_of(i)
                    pltpu.make_async_copy(
                        tab_hbm.at[:, pl.ds(0, slb * _BI)],
                        tb[b].at[:, pl.ds(0, slb * _BI)],
                        sg[b],
                    ).wait()

                    for s in range(slb):
                        @pl.when(i >= 1)
                        def _():
                            for p4 in range(4):
                                for q in range(2):
                                    pltpu.make_async_copy(
                                        tt[s].at[p4, q, :, pl.ds(0, 2 * d)],
                                        out_hbm.at[pl.ds(0, _CI)],
                                        so[s],
                                    ).wait()

                        @plsc.parallel_loop(0, d, unroll=2)
                        def _(c):
                            m_idx = (iota & 1) * d + c
                            for k in range(_BI // _LANES):
                                t_idx = iota + k * _LANES
                                v = tb[b][c, pl.ds(s * _BI + k * _LANES, _LANES)]
                                plsc.store_scatter(
                                    tt[s],
                                    [t_idx >> 5, (t_idx >> 4) & 1,
                                     (t_idx >> 1) & 7, m_idx],
                                    v,
                                )

                        for p4 in range(4):
                            for q in range(2):
                                row = pl.multiple_of(
                                    col0 // 2
                                    + s * (_BI // 2)
                                    + p4 * 16
                                    + q * _CI,
                                    _CI,
                                )
                                pltpu.async_copy(
                                    tt[s].at[p4, q, :, pl.ds(0, 2 * d)],
                                    out_hbm.at[pl.ds(row, _CI)],
                                    so[s],
                                )

                    @pl.when(i + _NBUF < n_mine)
                    def _():
                        pltpu.async_copy(
                            tab_hbm.at[:, pl.ds(col0_of(i + _NBUF), slb * _BI)],
                            tb[b].at[:, pl.ds(0, slb * _BI)],
                            sg[b],
                        )

            return carry

        lax.fori_loop(0, (n_mine + _NBUF - 1) // _NBUF, col_body, 0)

        for s in range(slb):
            @pl.when(n_mine > 0)
            def _():
                for p4 in range(4):
                    for q in range(2):
                        pltpu.make_async_copy(
                            tt[s].at[p4, q, :, pl.ds(0, 2 * d)],
                            out_hbm.at[pl.ds(0, _CI)],
                            so[s],
                        ).wait()

        if tail:
            @pl.when(w == _NW - 1)
            def _():
                for r in range(tail // 2 // _CI):
                    pltpu.sync_copy(
                        tail_hbm.at[pl.ds(r * _CI, _CI)],
                        tt[0].at[r, 0, :, pl.ds(0, 2 * d)],
                    )
                    pltpu.sync_copy(
                        tt[0].at[r, 0, :, pl.ds(0, 2 * d)],
                        out_hbm.at[
                            pl.ds(n_full * (_BI // 2) + r * _CI, _CI)
                        ],
                    )

    return run


def _lookup_kernel(seq_len, d, n_jb, scale):
    mesh = plsc.VectorSubcoreMesh(core_axis_name="c", subcore_axis_name="s")
    n_jc = d // _CI
    kvecs = d // _LANES

    @functools.partial(
        pl.kernel,
        mesh=mesh,
        out_type=jax.ShapeDtypeStruct((seq_len, n_jc, n_jb, _CI, _BI), jnp.float32),
        scratch_types=[
            pltpu.VMEM((seq_len, _BI), jnp.int32),
            [pltpu.VMEM((_BI, d), jnp.float32)] * _NBUF,
            [pltpu.VMEM((d, _BI + 1), jnp.float32)] * _NBUF,
            [pltpu.SemaphoreType.DMA] * _NBUF,
            [pltpu.SemaphoreType.DMA] * _NBUF,
        ],
        compiler_params=pltpu.CompilerParams(
            use_tc_tiling_on_sc=False, needs_layout_passes=False
        ),
    )
    def run(table_hbm, idx_hbm, out_hbm, idx_v, gb, ob, sg, so):
        cid = lax.axis_index("c")
        sid = lax.axis_index("s")
        w = sid * 2 + cid
        pltpu.sync_copy(idx_hbm.at[:, w], idx_v)

        iota = jnp.arange(_LANES, dtype=jnp.int32)

        for b in range(_NBUF):
            pltpu.async_copy(table_hbm.at[idx_v.at[b]], gb[b], sg[b])

        def group_body(p, carry):
            for b in range(_NBUF):
                j = p * _NBUF + b
                pltpu.make_async_copy(
                    table_hbm.at[pl.ds(0, _BI)], gb[b], sg[b]
                ).wait()

                @pl.when(p > 0)
                def _():
                    for jc in range(n_jc):
                        pltpu.make_async_copy(
                            ob[b].at[pl.ds(jc * _CI, _CI), pl.ds(0, _BI)],
                            out_hbm.at[0, jc, w],
                            so[b],
                        ).wait()

                @plsc.parallel_loop(0, _BI, unroll=2)
                def _(t):
                    t_vec = jnp.broadcast_to(t, (_LANES,))
                    for k in range(kvecs):
                        c_idx = iota + (k * _LANES)
                        v = gb[b][t, pl.ds(k * _LANES, _LANES)] * scale
                        plsc.store_scatter(ob[b], [c_idx, t_vec], v)

                for jc in range(n_jc):
                    pltpu.async_copy(
                        ob[b].at[pl.ds(jc * _CI, _CI), pl.ds(0, _BI)],
                        out_hbm.at[j, jc, w],
                        so[b],
                    )

                @pl.when(j + _NBUF < seq_len)
                def _():
                    pltpu.async_copy(
                        table_hbm.at[idx_v.at[j + _NBUF]], gb[b], sg[b]
                    )

            return carry

        lax.fori_loop(0, seq_len // _NBUF, group_body, 0)

        for b in range(_NBUF):
            for jc in range(n_jc):
                pltpu.make_async_copy(
                    ob[b].at[pl.ds(jc * _CI, _CI), pl.ds(0, _BI)],
                    out_hbm.at[0, jc, w],
                    so[b],
                ).wait()

    return run


def kernel(table, x):
    v, d = table.shape
    bsz, seq_len = x.shape
    n_jb = bsz // _BI
    scale = float(d) ** -0.5

    n_full = v // _BI
    tail_rows = table[n_full * _BI:].reshape(-1, 2 * d)
    table_lin = _detile_kernel(d, v)(table.T, tail_rows).reshape(v, d)

    idx = x.T.reshape(seq_len, n_jb, _BI)
    out5 = _lookup_kernel(seq_len, d, n_jb, scale)(table_lin, idx)
    out = out5.transpose(2, 4, 0, 1, 3).reshape(bsz, seq_len, d)
    return out


# speedup vs baseline: 1.3509x; 1.3509x over previous
import functools

import jax
import jax.numpy as jnp
from jax import lax
from jax.experimental import pallas as pl
from jax.experimental.pallas import tpu as pltpu
from jax.experimental.pallas import tpu_sc as plsc

_LANES = 16
_BI = 128
_CI = 8
_NBUF = 2


def _embed_kernel(seq_len, d, n_jb, scale):
    mesh = plsc.VectorSubcoreMesh(core_axis_name="c", subcore_axis_name="s")
    n_jc = d // _CI

    @functools.partial(
        pl.kernel,
        mesh=mesh,
        out_type=jax.ShapeDtypeStruct((seq_len, n_jc, n_jb, _CI, _BI), jnp.float32),
        scratch_types=[
            pltpu.VMEM((seq_len, _BI), jnp.int32),
            [pltpu.VMEM((_BI, d), jnp.float32)] * _NBUF,
            [pltpu.VMEM((d, _BI + 1), jnp.float32)] * _NBUF,
            [pltpu.SemaphoreType.DMA] * _NBUF,
            [pltpu.SemaphoreType.DMA] * _NBUF,
        ],
        compiler_params=pltpu.CompilerParams(
            use_tc_tiling_on_sc=False, needs_layout_passes=False
        ),
    )
    def run(table_hbm, idx_hbm, out_hbm, idx_v, gb, ob, sg, so):
        cid = lax.axis_index("c")
        sid = lax.axis_index("s")
        w = sid * 2 + cid
        pltpu.sync_copy(idx_hbm.at[:, w], idx_v)

        iota = jnp.arange(_LANES, dtype=jnp.int32)

        for b in range(_NBUF):
            pltpu.async_copy(table_hbm.at[idx_v.at[b]], gb[b], sg[b])

        def group_body(p, carry):
            for b in range(_NBUF):
                j = p * _NBUF + b
                pltpu.make_async_copy(
                    table_hbm.at[pl.ds(0, _BI)], gb[b], sg[b]
                ).wait()

                @pl.when(p > 0)
                def _():
                    for jc in range(n_jc):
                        pltpu.make_async_copy(
                            ob[b].at[pl.ds(jc * _CI, _CI), pl.ds(0, _BI)],
                            out_hbm.at[0, jc, w],
                            so[b],
                        ).wait()

                @plsc.parallel_loop(0, _BI, unroll=2)
                def _(t):
                    t_vec = jnp.broadcast_to(t, (_LANES,))
                    for k in range(d // _LANES):
                        c_idx = iota + (k * _LANES)
                        v = gb[b][t, pl.ds(k * _LANES, _LANES)] * scale
                        plsc.store_scatter(ob[b], [c_idx, t_vec], v)

                for jc in range(n_jc):
                    pltpu.async_copy(
                        ob[b].at[pl.ds(jc * _CI, _CI), pl.ds(0, _BI)],
                        out_hbm.at[j, jc, w],
                        so[b],
                    )

                @pl.when(j + _NBUF < seq_len)
                def _():
                    pltpu.async_copy(
                        table_hbm.at[idx_v.at[j + _NBUF]], gb[b], sg[b]
                    )

            return carry

        lax.fori_loop(0, seq_len // _NBUF, group_body, 0)

        for b in range(_NBUF):
            for jc in range(n_jc):
                pltpu.make_async_copy(
                    ob[b].at[pl.ds(jc * _CI, _CI), pl.ds(0, _BI)],
                    out_hbm.at[0, jc, w],
                    so[b],
                ).wait()

    return run


def kernel(table, x):
    v, d = table.shape
    bsz, seq_len = x.shape
    n_jb = bsz // _BI
    scale = float(d) ** -0.5

    idx = x.T.reshape(seq_len, n_jb, _BI)
    out5 = _embed_kernel(seq_len, d, n_jb, scale)(table, idx)
    out = out5.transpose(2, 4, 0, 1, 3).reshape(bsz, seq_len, d)
    return out
